# baseline (speedup 1.0000x reference)
import functools

import jax
import jax.numpy as jnp
from jax import lax
from jax.experimental import pallas as pl
from jax.experimental.pallas import tpu as pltpu
from jax.experimental.pallas import tpu_sc as plsc

N_ENT = 1000000
N_FACTORS = 16
BATCH = 16384

NUM_CORES = 2
NUM_SUBCORES = 16
NUM_WORKERS = NUM_CORES * NUM_SUBCORES
B_PER_W = BATCH // NUM_WORKERS
CHUNK = 128
N_CHUNKS = B_PER_W // CHUNK
N_GROUPS = B_PER_W // N_FACTORS


def _sc_body(heads_hbm, tails_hbm, emb_h_hbm, emb_t_hbm, out_hbm,
             idx_h, idx_t, rows_h, rows_t, out_v, sem):
    wid = lax.axis_index("s") * NUM_CORES + lax.axis_index("c")

    pltpu.sync_copy(heads_hbm.at[pl.ds(wid * N_CHUNKS, N_CHUNKS)], idx_h)
    pltpu.sync_copy(tails_hbm.at[pl.ds(wid * N_CHUNKS, N_CHUNKS)], idx_t)

    copies = []
    for j in range(N_CHUNKS):
        copies.append(pltpu.async_copy(
            emb_h_hbm.at[idx_h.at[j]], rows_h.at[pl.ds(j * CHUNK, CHUNK)], sem))
        copies.append(pltpu.async_copy(
            emb_t_hbm.at[idx_t.at[j]], rows_t.at[pl.ds(j * CHUNK, CHUNK)], sem))
    for c in copies:
        c.wait()

    lane = lax.iota(jnp.int32, 16)

    def group(g, carry):
        row0 = g * N_FACTORS
        rows = row0 + lane
        acc = jnp.zeros((16,), jnp.float32)
        for d in range(N_FACTORS):
            dv = jnp.full((16,), d, jnp.int32)
            hv = plsc.load_gather(rows_h, [rows, dv])
            tv = plsc.load_gather(rows_t, [rows, dv])
            acc = acc + hv * tv
        out_v[pl.ds(row0, 16)] = acc
        return carry

    lax.fori_loop(0, N_GROUPS, group, 0)

    pltpu.sync_copy(out_v, out_hbm.at[pl.ds(wid * B_PER_W, B_PER_W)])


@jax.jit
def _run(heads2d, tails2d, emb_heads, emb_tails):
    mesh = plsc.VectorSubcoreMesh(core_axis_name="c", subcore_axis_name="s")
    f = pl.kernel(
        _sc_body,
        mesh=mesh,
        compiler_params=pltpu.CompilerParams(
            needs_layout_passes=False, use_tc_tiling_on_sc=False),
        out_type=jax.ShapeDtypeStruct((BATCH,), jnp.float32),
        scratch_types=[
            pltpu.VMEM((N_CHUNKS, CHUNK), jnp.int32),
            pltpu.VMEM((N_CHUNKS, CHUNK), jnp.int32),
            pltpu.VMEM((B_PER_W, N_FACTORS), jnp.float32),
            pltpu.VMEM((B_PER_W, N_FACTORS), jnp.float32),
            pltpu.VMEM((B_PER_W,), jnp.float32),
            pltpu.SemaphoreType.DMA,
        ],
    )
    return f(heads2d, tails2d, emb_heads, emb_tails)


def kernel(heads, tails, emb_heads, emb_tails):
    heads2d = heads.astype(jnp.int32).reshape(NUM_WORKERS * N_CHUNKS, CHUNK)
    tails2d = tails.astype(jnp.int32).reshape(NUM_WORKERS * N_CHUNKS, CHUNK)
    return _run(heads2d, tails2d, emb_heads, emb_tails)
